# baseline (speedup 1.0000x reference)
import jax
import jax.numpy as jnp
from jax.experimental import pallas as pl
from jax.experimental.pallas import tpu as pltpu

NUM_TOKENS = 8192
D_MODEL = 4096
NUM_EXPERTS = 64
BLOCK_M = 512
HALF = BLOCK_M // 2
NUM_BLOCKS = NUM_TOKENS // BLOCK_M
NBUF = 3


def _router_body(x_hbm, w_ref, o_hbm, buf_ref, out_ref, in_sems, out_sems):
    def half_copy(i, h):
        slot = i % NBUF
        return pltpu.make_async_copy(
            x_hbm.at[pl.ds(i * BLOCK_M + h * HALF, HALF), :],
            buf_ref.at[slot, pl.ds(h * HALF, HALF), :],
            in_sems.at[slot, h],
        )

    def start_block(i):
        half_copy(i, 0).start()
        half_copy(i, 1).start()

    def wait_block(i):
        half_copy(i, 0).wait()
        half_copy(i, 1).wait()

    def out_copy(i):
        slot = i % NBUF
        return pltpu.make_async_copy(
            out_ref.at[slot],
            o_hbm.at[:, pl.ds(i * BLOCK_M, BLOCK_M)],
            out_sems.at[slot],
        )

    for i in range(NBUF):
        start_block(i)
    for i in range(NUM_BLOCKS):
        slot = i % NBUF
        wait_block(i)
        if i >= NBUF:
            out_copy(i - NBUF).wait()
        out_ref[slot] = jax.lax.dot_general(
            w_ref[...],
            buf_ref[slot],
            (((1,), (1,)), ((), ())),
            preferred_element_type=jnp.float32,
        )
        out_copy(i).start()
        if i + NBUF < NUM_BLOCKS:
            start_block(i + NBUF)
    for i in range(NUM_BLOCKS - NBUF, NUM_BLOCKS):
        out_copy(i).wait()


@jax.jit
def kernel(x, W):
    out_t = pl.pallas_call(
        _router_body,
        in_specs=[
            pl.BlockSpec(memory_space=pltpu.MemorySpace.HBM),
            pl.BlockSpec(memory_space=pltpu.MemorySpace.VMEM),
        ],
        out_specs=pl.BlockSpec(memory_space=pltpu.MemorySpace.HBM),
        out_shape=jax.ShapeDtypeStruct((NUM_EXPERTS, NUM_TOKENS), jnp.float32),
        scratch_shapes=[
            pltpu.VMEM((NBUF, BLOCK_M, D_MODEL), jnp.float32),
            pltpu.VMEM((NBUF, NUM_EXPERTS, BLOCK_M), jnp.float32),
            pltpu.SemaphoreType.DMA((NBUF, 2)),
            pltpu.SemaphoreType.DMA((NBUF,)),
        ],
        compiler_params=pltpu.CompilerParams(
            vmem_limit_bytes=100 * 1024 * 1024,
        ),
    )(x, W)
    return out_t.T
